# Initial kernel scaffold; baseline (speedup 1.0000x reference)
import jax
import jax.numpy as jnp
from jax.experimental import pallas as pl


def kernel(x, edge_index, W1, b1, W2, b2, W3, b3, W4, b4):
    raise NotImplementedError("write your pallas kernel here")


# speedup vs baseline: 6.5896x; 6.5896x over previous
import functools

import jax
import jax.numpy as jnp
from jax import lax
from jax.experimental import pallas as pl
from jax.experimental.pallas import tpu as pltpu
from jax.experimental.pallas import tpu_sc as plsc

N = 10000
E = 320000
D = 128
NC, NS = 2, 16
NW = NC * NS
EROWS = -(-E // 128)
ROWS_PER_TILE = ((-(-EROWS // NW)) + 7) // 8 * 8
EPAD_ROWS = ROWS_PER_TILE * NW
IDX_CHUNK = 8
DUMP = N
ACC_ROWS = 10240
ZROWS = ACC_ROWS // NS
DEGW = 128

@functools.lru_cache(maxsize=None)
def _sc_kernels():
    mesh = plsc.VectorSubcoreMesh(core_axis_name="c", subcore_axis_name="s",
                                  num_cores=NC, num_subcores=NS)

    @functools.partial(
        pl.kernel,
        out_type=jax.ShapeDtypeStruct((NC, ACC_ROWS, DEGW), jnp.float32),
        mesh=mesh,
        scratch_types=[
            pltpu.VMEM((IDX_CHUNK, 128), jnp.int32),
            pltpu.VMEM((128, DEGW), jnp.float32),
            pltpu.VMEM_SHARED((ACC_ROWS, DEGW), jnp.float32),
        ],
    )
    def sc_degree(dst_hbm, z_hbm, ones_hbm, out_hbm, dstb, onesb, acc):
        cid = lax.axis_index("c")
        sid = lax.axis_index("s")
        wid = sid * NC + cid
        pltpu.sync_copy(z_hbm, acc.at[pl.ds(sid * ZROWS, ZROWS)])
        pltpu.sync_copy(ones_hbm, onesb)
        plsc.subcore_barrier()
        base = wid * ROWS_PER_TILE

        def body(i, c):
            pltpu.sync_copy(dst_hbm.at[pl.ds(base + i * IDX_CHUNK, IDX_CHUNK)],
                            dstb)
            for j in range(IDX_CHUNK):
                pltpu.sync_copy(onesb, acc.at[dstb.at[j]], add=True)
            return c

        lax.fori_loop(0, ROWS_PER_TILE // IDX_CHUNK, body, 0)
        plsc.subcore_barrier()
        pltpu.sync_copy(acc.at[pl.ds(sid * ZROWS, ZROWS)],
                        out_hbm.at[cid, pl.ds(sid * ZROWS, ZROWS)])

    @functools.partial(
        pl.kernel,
        out_type=jax.ShapeDtypeStruct((NC, ACC_ROWS, D), jnp.float32),
        mesh=mesh,
        scratch_types=[
            pltpu.VMEM((IDX_CHUNK, 128), jnp.int32),
            pltpu.VMEM((IDX_CHUNK, 128), jnp.int32),
            pltpu.VMEM((128, D), jnp.float32),
            pltpu.VMEM_SHARED((ACC_ROWS, D), jnp.float32),
            pltpu.SemaphoreType.DMA,
        ],
    )
    def sc_aggregate(g_hbm, src_hbm, dst_hbm, z_hbm, out_hbm,
                     srcb, dstb, rows, acc, sem):
        cid = lax.axis_index("c")
        sid = lax.axis_index("s")
        wid = sid * NC + cid
        pltpu.sync_copy(z_hbm, acc.at[pl.ds(sid * ZROWS, ZROWS)])
        plsc.subcore_barrier()
        base = wid * ROWS_PER_TILE

        def body(i, c):
            pltpu.sync_copy(src_hbm.at[pl.ds(base + i * IDX_CHUNK, IDX_CHUNK)],
                            srcb)
            pltpu.sync_copy(dst_hbm.at[pl.ds(base + i * IDX_CHUNK, IDX_CHUNK)],
                            dstb)
            for j in range(IDX_CHUNK):
                pltpu.async_copy(g_hbm.at[srcb.at[j]], rows, sem).wait()
                pltpu.sync_copy(rows, acc.at[dstb.at[j]], add=True)
            return c

        lax.fori_loop(0, ROWS_PER_TILE // IDX_CHUNK, body, 0)
        plsc.subcore_barrier()
        pltpu.sync_copy(acc.at[pl.ds(sid * ZROWS, ZROWS)],
                        out_hbm.at[cid, pl.ds(sid * ZROWS, ZROWS)])

    return sc_degree, sc_aggregate


def _sc_degree(dst2d, z, ones):
    return _sc_kernels()[0](dst2d, z, ones)


def _sc_aggregate(g, src2d, dst2d, z):
    return _sc_kernels()[1](g, src2d, dst2d, z)


BLK = 1000


def _tc_layer1(deg_parts, x, W1):
    def body(deg_ref, x_ref, w_ref, dinv_ref, g_ref):
        dp = deg_ref[...]
        deg = dp[0, :, 0:1] + dp[1, :, 0:1] + 1.0
        dinv_b = jnp.broadcast_to(lax.rsqrt(deg), (BLK, D))
        dinv_ref[...] = dinv_b
        g_ref[...] = jnp.dot(x_ref[...], w_ref[...],
                             preferred_element_type=jnp.float32) * dinv_b

    return pl.pallas_call(
        body,
        grid=(N // BLK,),
        in_specs=[
            pl.BlockSpec((NC, BLK, DEGW), lambda i: (0, i, 0)),
            pl.BlockSpec((BLK, D), lambda i: (i, 0)),
            pl.BlockSpec((D, D), lambda i: (0, 0)),
        ],
        out_specs=[
            pl.BlockSpec((BLK, D), lambda i: (i, 0)),
            pl.BlockSpec((BLK, D), lambda i: (i, 0)),
        ],
        out_shape=[
            jax.ShapeDtypeStruct((N, D), jnp.float32),
            jax.ShapeDtypeStruct((N, D), jnp.float32),
        ],
    )(deg_parts, x, W1)


def _tc_mid(p, g_prev, dinv_b, b, W):
    def body(p_ref, gp_ref, dv_ref, b_ref, w_ref, out_ref):
        agg = p_ref[0] + p_ref[1] + gp_ref[...]
        t = jnp.maximum(dv_ref[...] * agg + b_ref[...], 0.0)
        out_ref[...] = jnp.dot(t, w_ref[...],
                               preferred_element_type=jnp.float32) * dv_ref[...]

    return pl.pallas_call(
        body,
        grid=(N // BLK,),
        in_specs=[
            pl.BlockSpec((NC, BLK, D), lambda i: (0, i, 0)),
            pl.BlockSpec((BLK, D), lambda i: (i, 0)),
            pl.BlockSpec((BLK, D), lambda i: (i, 0)),
            pl.BlockSpec((1, D), lambda i: (0, 0)),
            pl.BlockSpec((D, D), lambda i: (0, 0)),
        ],
        out_specs=pl.BlockSpec((BLK, D), lambda i: (i, 0)),
        out_shape=jax.ShapeDtypeStruct((N, D), jnp.float32),
    )(p, g_prev, dinv_b, b, W)


def _tc_final(p, g_prev, dinv_b, b, Wp, b4p):
    def body(p_ref, gp_ref, dv_ref, b_ref, w_ref, b4_ref, out_ref):
        agg = p_ref[0] + p_ref[1] + gp_ref[...]
        t = jnp.maximum(dv_ref[...] * agg + b_ref[...], 0.0)
        out_ref[...] = jnp.dot(t, w_ref[...],
                               preferred_element_type=jnp.float32) + b4_ref[...]

    return pl.pallas_call(
        body,
        grid=(N // BLK,),
        in_specs=[
            pl.BlockSpec((NC, BLK, D), lambda i: (0, i, 0)),
            pl.BlockSpec((BLK, D), lambda i: (i, 0)),
            pl.BlockSpec((BLK, D), lambda i: (i, 0)),
            pl.BlockSpec((1, D), lambda i: (0, 0)),
            pl.BlockSpec((D, D), lambda i: (0, 0)),
            pl.BlockSpec((1, D), lambda i: (0, 0)),
        ],
        out_specs=pl.BlockSpec((BLK, D), lambda i: (i, 0)),
        out_shape=jax.ShapeDtypeStruct((N, D), jnp.float32),
    )(p, g_prev, dinv_b, b, Wp, b4p)


def kernel(x, edge_index, W1, b1, W2, b2, W3, b3, W4, b4):
    src = edge_index[0]
    dst = edge_index[1]
    pad = EPAD_ROWS * 128 - E
    src2d = jnp.concatenate(
        [src, jnp.zeros((pad,), jnp.int32)]).reshape(EPAD_ROWS, 128)
    dst2d = jnp.concatenate(
        [dst, jnp.full((pad,), DUMP, jnp.int32)]).reshape(EPAD_ROWS, 128)
    zeros_deg = jnp.zeros((ZROWS, DEGW), jnp.float32)
    ones_row = jnp.zeros((128, DEGW), jnp.float32).at[:, 0].set(1.0)
    zeros_rows = jnp.zeros((ZROWS, D), jnp.float32)
    b1r = b1.reshape(1, D)
    b2r = b2.reshape(1, D)
    b3r = b3.reshape(1, D)
    W4p = jnp.zeros((D, D), jnp.float32).at[:, :W4.shape[1]].set(W4)
    b4p = jnp.zeros((1, D), jnp.float32).at[0, :b4.shape[0]].set(b4)

    deg_parts = _sc_degree(dst2d, zeros_deg, ones_row)
    dinv_b, g1 = _tc_layer1(deg_parts, x, W1)
    p1 = _sc_aggregate(g1, src2d, dst2d, zeros_rows)
    g2 = _tc_mid(p1, g1, dinv_b, b1r, W2)
    p2 = _sc_aggregate(g2, src2d, dst2d, zeros_rows)
    g3 = _tc_mid(p2, g2, dinv_b, b2r, W3)
    p3 = _sc_aggregate(g3, src2d, dst2d, zeros_rows)
    out = _tc_final(p3, g3, dinv_b, b3r, W4p, b4p)
    return out[:, :3]
